# Initial kernel scaffold; baseline (speedup 1.0000x reference)
import jax
import jax.numpy as jnp
from jax.experimental import pallas as pl


def kernel(x, parent_embeddings):
    raise NotImplementedError("write your pallas kernel here")


# speedup vs baseline: 3.3269x; 3.3269x over previous
import functools

import jax
import jax.numpy as jnp
from jax import lax
from jax.experimental import pallas as pl
from jax.experimental.pallas import tpu as pltpu
from jax.experimental.pallas import tpu_sc as plsc

D_MODEL = 128
CHUNK = 128
NBUF = 2


def _body(idx_hbm, table_hbm, out_hbm, idx_v, rows_v, gsem0, gsem1):
    info = plsc.get_sparse_core_info()
    nc = info.num_cores
    wid = lax.axis_index("s") * nc + lax.axis_index("c")
    n_ch = idx_v.shape[0]
    base = wid * (n_ch * CHUNK)
    sems = [gsem0, gsem1]

    pltpu.sync_copy(idx_hbm.at[wid], idx_v)

    def _add1(r, _):
        for c in range(CHUNK // 16):
            sl = pl.ds(c * 16, 16)
            idx_v[r, sl] = idx_v[r, sl] + 1
        return 0

    lax.fori_loop(0, n_ch, _add1, 0)

    for b in range(NBUF):
        pltpu.async_copy(table_hbm.at[idx_v.at[b]], rows_v.at[b], sems[b])

    def _step(g, _):
        for b in range(NBUF):
            j = g * NBUF + b
            pltpu.make_async_copy(
                table_hbm.at[idx_v.at[b]], rows_v.at[b], sems[b]
            ).wait()
            pltpu.sync_copy(
                rows_v.at[b], out_hbm.at[pl.ds(base + j * CHUNK, CHUNK)]
            )

            @pl.when(j + NBUF < n_ch)
            def _():
                pltpu.async_copy(
                    table_hbm.at[idx_v.at[j + NBUF]], rows_v.at[b], sems[b]
                )

        return 0

    lax.fori_loop(0, n_ch // NBUF, _step, 0)


def kernel(x, parent_embeddings):
    batch, seq = x.shape
    n = batch * seq
    info = plsc.get_sparse_core_info()
    nw = info.num_cores * info.num_subcores
    n_per_w = n // nw
    n_ch = n_per_w // CHUNK
    idx = x.astype(jnp.int32).reshape(nw, n_ch, CHUNK)

    mesh = plsc.VectorSubcoreMesh(core_axis_name="c", subcore_axis_name="s")
    run = functools.partial(
        pl.kernel,
        mesh=mesh,
        out_type=jax.ShapeDtypeStruct((n, D_MODEL), jnp.float32),
        scratch_types=[
            pltpu.VMEM((n_ch, CHUNK), jnp.int32),
            pltpu.VMEM((NBUF, CHUNK, D_MODEL), jnp.float32),
            pltpu.SemaphoreType.DMA,
            pltpu.SemaphoreType.DMA,
        ],
    )(_body)
    out = run(idx, parent_embeddings)
    return out.reshape(batch, seq, D_MODEL)


# speedup vs baseline: 3.3269x; 1.0000x over previous
import functools

import jax
import jax.numpy as jnp
from jax import lax
from jax.experimental import pallas as pl
from jax.experimental.pallas import tpu as pltpu
from jax.experimental.pallas import tpu_sc as plsc

D_MODEL = 128
CHUNK = 128
NBUF = 5
LAG = 2


def _body(idx_hbm, table_hbm, out_hbm, idx_v, rows_v, *sems):
    gsems = list(sems[:NBUF])
    wsems = list(sems[NBUF:])
    info = plsc.get_sparse_core_info()
    nc = info.num_cores
    wid = lax.axis_index("s") * nc + lax.axis_index("c")
    n_ch = idx_v.shape[0]
    base = wid * (n_ch * CHUNK)

    def g_start(j, b):
        pltpu.async_copy(table_hbm.at[idx_v.at[j]], rows_v.at[b], gsems[b])

    def g_wait(j, b):
        pltpu.make_async_copy(
            table_hbm.at[idx_v.at[j]], rows_v.at[b], gsems[b]
        ).wait()

    def w_start(j, b):
        pltpu.async_copy(
            rows_v.at[b], out_hbm.at[pl.ds(base + j * CHUNK, CHUNK)], wsems[b]
        )

    def w_wait(j, b):
        pltpu.make_async_copy(
            rows_v.at[b], out_hbm.at[pl.ds(base + j * CHUNK, CHUNK)], wsems[b]
        ).wait()

    pltpu.sync_copy(idx_hbm.at[wid], idx_v)

    def _add1(r, _):
        for c in range(CHUNK // 16):
            sl = pl.ds(c * 16, 16)
            idx_v[r, sl] = idx_v[r, sl] + 1
        return 0

    lax.fori_loop(0, n_ch, _add1, 0)

    for j in range(LAG):
        g_start(j, j % NBUF)

    def _step(g, _):
        for b in range(NBUF):
            j = g * NBUF + b
            g_wait(j, b)
            w_start(j, b)

            @pl.when(j >= NBUF - LAG)
            def _():
                w_wait(j - (NBUF - LAG), (b + LAG) % NBUF)

            @pl.when(j + LAG < n_ch)
            def _():
                g_start(j + LAG, (b + LAG) % NBUF)

        return 0

    lax.fori_loop(0, n_ch // NBUF, _step, 0)

    for j in range(n_ch - (NBUF - LAG), n_ch):
        w_wait(j, j % NBUF)


def kernel(x, parent_embeddings):
    batch, seq = x.shape
    n = batch * seq
    info = plsc.get_sparse_core_info()
    nw = info.num_cores * info.num_subcores
    n_per_w = n // nw
    n_ch = n_per_w // CHUNK
    idx = x.astype(jnp.int32).reshape(nw, n_ch, CHUNK)

    mesh = plsc.VectorSubcoreMesh(core_axis_name="c", subcore_axis_name="s")
    run = functools.partial(
        pl.kernel,
        mesh=mesh,
        out_type=jax.ShapeDtypeStruct((n, D_MODEL), jnp.float32),
        scratch_types=[
            pltpu.VMEM((n_ch, CHUNK), jnp.int32),
            pltpu.VMEM((NBUF, CHUNK, D_MODEL), jnp.float32),
        ]
        + [pltpu.SemaphoreType.DMA] * (2 * NBUF),
    )(_body)
    out = run(idx, parent_embeddings)
    return out.reshape(batch, seq, D_MODEL)


# speedup vs baseline: 5.9251x; 1.7810x over previous
import functools

import jax
import jax.numpy as jnp
from jax import lax
from jax.experimental import pallas as pl
from jax.experimental.pallas import tpu as pltpu
from jax.experimental.pallas import tpu_sc as plsc

D_MODEL = 128
NB = 2
NBUF = 4
LAG = 2


def _body(idx_hbm, table_hbm, out_hbm, idx_v, rows_v, *sems):
    gsems = list(sems[:NBUF])
    wsems = list(sems[NBUF:])
    info = plsc.get_sparse_core_info()
    nc = info.num_cores
    wid = lax.axis_index("s") * nc + lax.axis_index("c")
    n_ch = idx_v.shape[0]
    b0 = wid * (n_ch * NB)

    def g_start(j, b):
        pltpu.async_copy(table_hbm.at[idx_v.at[j]], rows_v.at[b], gsems[b])

    def g_wait(j, b):
        pltpu.make_async_copy(
            table_hbm.at[idx_v.at[j]], rows_v.at[b], gsems[b]
        ).wait()

    seq_l = idx_v.shape[1] // NB

    def w_start(j, b):
        for r in range(NB):
            pltpu.async_copy(
                rows_v.at[b, pl.ds(r * seq_l, seq_l)],
                out_hbm.at[b0 + j * NB + r],
                wsems[b],
            )

    def w_wait(j, b):
        for r in range(NB):
            pltpu.make_async_copy(
                rows_v.at[b, pl.ds(r * seq_l, seq_l)],
                out_hbm.at[b0 + j * NB + r],
                wsems[b],
            ).wait()

    pltpu.sync_copy(idx_hbm.at[wid], idx_v)

    row_w = idx_v.shape[1]
    n_full = row_w // 16
    lanes = lax.iota(jnp.int32, 16)

    def _add1(r, _):
        for c in range(n_full):
            sl = pl.ds(c * 16, 16)
            idx_v[r, sl] = idx_v[r, sl] + 1
        if row_w % 16:
            sl = pl.ds(row_w - 16, 16)
            tail = idx_v[r, sl]
            idx_v[r, sl] = jnp.where(
                lanes >= (16 - row_w % 16), tail + 1, tail
            )
        return 0

    lax.fori_loop(0, n_ch, _add1, 0)

    for j in range(LAG):
        g_start(j, j % NBUF)

    def _step(g, _):
        for b in range(NBUF):
            j = g * NBUF + b
            g_wait(j, b)
            w_start(j, b)

            @pl.when(j >= NBUF - LAG)
            def _():
                w_wait(j - (NBUF - LAG), (b + LAG) % NBUF)

            @pl.when(j + LAG < n_ch)
            def _():
                g_start(j + LAG, (b + LAG) % NBUF)

        return 0

    lax.fori_loop(0, n_ch // NBUF, _step, 0)

    for j in range(n_ch - (NBUF - LAG), n_ch):
        w_wait(j, j % NBUF)


def kernel(x, parent_embeddings):
    batch, seq = x.shape
    n = batch * seq
    info = plsc.get_sparse_core_info()
    nw = info.num_cores * info.num_subcores
    n_ch = batch // (nw * NB)
    idx = x.astype(jnp.int32).reshape(nw, n_ch, NB * seq)

    mesh = plsc.VectorSubcoreMesh(core_axis_name="c", subcore_axis_name="s")
    run = functools.partial(
        pl.kernel,
        mesh=mesh,
        out_type=jax.ShapeDtypeStruct((batch, seq, D_MODEL), jnp.float32),
        scratch_types=[
            pltpu.VMEM((n_ch, NB * seq), jnp.int32),
            pltpu.VMEM((NBUF, NB * seq, D_MODEL), jnp.float32),
        ]
        + [pltpu.SemaphoreType.DMA] * (2 * NBUF),
    )(_body)
    return run(idx, parent_embeddings)


# speedup vs baseline: 10.6593x; 1.7990x over previous
import functools

import jax
import jax.numpy as jnp
from jax import lax
from jax.experimental import pallas as pl
from jax.experimental.pallas import tpu as pltpu
from jax.experimental.pallas import tpu_sc as plsc

D_MODEL = 128
CHUNK = 128
NBUF = 5
LAG = 2


def _body(idx_hbm, table_hbm, out_hbm, idx_v, rows_v, *sems):
    gsems = list(sems[:NBUF])
    wsems = list(sems[NBUF:])
    info = plsc.get_sparse_core_info()
    nc = info.num_cores
    wid = lax.axis_index("s") * nc + lax.axis_index("c")
    n_ch = idx_v.shape[0]
    b0 = wid * CHUNK

    def g_start(j, b):
        pltpu.async_copy(table_hbm.at[idx_v.at[j]], rows_v.at[b], gsems[b])

    def g_wait(j, b):
        pltpu.make_async_copy(
            table_hbm.at[idx_v.at[j]], rows_v.at[b], gsems[b]
        ).wait()

    def w_start(j, b):
        pltpu.async_copy(
            rows_v.at[b], out_hbm.at[j, pl.ds(b0, CHUNK)], wsems[b]
        )

    def w_wait(j, b):
        pltpu.make_async_copy(
            rows_v.at[b], out_hbm.at[j, pl.ds(b0, CHUNK)], wsems[b]
        ).wait()

    pltpu.sync_copy(idx_hbm.at[:, pl.ds(b0, CHUNK)], idx_v)

    def _add1(r, _):
        for c in range(CHUNK // 16):
            sl = pl.ds(c * 16, 16)
            idx_v[r, sl] = idx_v[r, sl] + 1
        return 0

    lax.fori_loop(0, n_ch, _add1, 0)

    for j in range(LAG):
        g_start(j, j % NBUF)

    def _step(g, _):
        for b in range(NBUF):
            j = g * NBUF + b
            g_wait(j, b)
            w_start(j, b)

            @pl.when(j >= NBUF - LAG)
            def _():
                w_wait(j - (NBUF - LAG), (b + LAG) % NBUF)

            @pl.when(j + LAG < n_ch)
            def _():
                g_start(j + LAG, (b + LAG) % NBUF)

        return 0

    lax.fori_loop(0, n_ch // NBUF, _step, 0)

    for j in range(n_ch - (NBUF - LAG), n_ch):
        w_wait(j, j % NBUF)


def kernel(x, parent_embeddings):
    batch, seq = x.shape
    idx = x.T.astype(jnp.int32)

    mesh = plsc.VectorSubcoreMesh(core_axis_name="c", subcore_axis_name="s")
    run = functools.partial(
        pl.kernel,
        mesh=mesh,
        out_type=jax.ShapeDtypeStruct((seq, batch, D_MODEL), jnp.float32),
        scratch_types=[
            pltpu.VMEM((seq, CHUNK), jnp.int32),
            pltpu.VMEM((NBUF, CHUNK, D_MODEL), jnp.float32),
        ]
        + [pltpu.SemaphoreType.DMA] * (2 * NBUF),
    )(_body)
    out = run(idx, parent_embeddings)
    return jnp.transpose(out, (1, 0, 2))
